# baseline (speedup 1.0000x reference)
import functools

import jax
import jax.numpy as jnp
from jax import lax
from jax.experimental import pallas as pl
from jax.experimental.pallas import tpu as pltpu
from jax.experimental.pallas import tpu_sc as plsc

MASK_TOKEN_ID = 99999
NATIVE_DRAFT_LEN = 8

try:
    _info = plsc.get_sparse_core_info()
    _NC = _info.num_cores
    _NS = _info.num_subcores
except ValueError:
    _NC, _NS = 2, 16
_NW = _NC * _NS


@functools.cache
def _make_body(bsz, seqlen, hidden):
    t = seqlen + NATIVE_DRAFT_LEN - 1
    b_per_w = bsz // _NW
    mesh = plsc.VectorSubcoreMesh(
        core_axis_name="c",
        subcore_axis_name="s",
        num_cores=_NC,
        num_subcores=_NS,
    )
    nsem = 2 * b_per_w + 2

    @functools.partial(
        pl.kernel,
        mesh=mesh,
        out_type=jax.ShapeDtypeStruct((bsz * t, hidden), jnp.float32),
        scratch_types=[
            pltpu.VMEM((b_per_w, seqlen), jnp.int32),
            pltpu.VMEM((b_per_w, seqlen), jnp.int32),
            pltpu.VMEM((8,), jnp.int32),
            pltpu.VMEM((b_per_w, seqlen), jnp.int32),
            pltpu.VMEM((b_per_w, 8), jnp.int32),
            pltpu.VMEM((b_per_w * seqlen, hidden), jnp.float32),
            pltpu.VMEM((8, hidden), jnp.float32),
            pltpu.VMEM_SHARED((8, hidden), jnp.float32),
            pltpu.SemaphoreType.DMA,
        ]
        + [pltpu.SemaphoreType.DMA] * nsem,
    )
    def body(
        x_hbm, table_hbm, midx_hbm, mdst_hbm, out_hbm,
        x_v, gidx_v, gidx_m, didx_v, didx_m, rows_v, mask_v, mask_s,
        xsem, *sems,
    ):
        sid = lax.axis_index("s")
        wid = sid * _NC + lax.axis_index("c")
        base = wid * b_per_w
        xcopy = pltpu.async_copy(x_hbm.at[pl.ds(base, b_per_w)], x_v, xsem)

        @pl.when(sid == 0)
        def _fetch_mask():
            pltpu.sync_copy(midx_hbm, gidx_m)
            pltpu.async_copy(
                table_hbm.at[gidx_m], mask_v, sems[b_per_w]
            ).wait()
            pltpu.sync_copy(mask_v, mask_s)

        xcopy.wait()
        for b in range(b_per_w):
            gidx_v[b, :] = x_v[b, :]
        g_tok = [
            pltpu.async_copy(
                table_hbm.at[gidx_v.at[b].at[pl.ds(h * 8, 8)]],
                rows_v.at[pl.ds(b * seqlen + h * 8, 8)],
                sems[b],
            )
            for b in range(b_per_w)
            for h in range(seqlen // 8)
        ]
        iota = lax.iota(jnp.int32, seqlen)
        for b in range(b_per_w):
            didx_v[b, :] = (base + b) * t + iota
        pltpu.sync_copy(mdst_hbm.at[wid], didx_m)
        plsc.subcore_barrier()

        @pl.when(sid != 0)
        def _pull_mask():
            pltpu.sync_copy(mask_s, mask_v)

        scatters = []
        nh = seqlen // 8
        for b in range(b_per_w):
            for h in range(nh):
                g_tok[b * nh + h].wait()
            scatters.append(
                pltpu.async_copy(
                    rows_v.at[pl.ds(b * seqlen, seqlen)],
                    out_hbm.at[didx_v.at[b]],
                    sems[b_per_w + 1 + b],
                )
            )
        scatters.append(
            pltpu.async_copy(
                mask_v, out_hbm.at[didx_m.at[0]], sems[2 * b_per_w + 1]
            )
        )
        for b in range(1, b_per_w):
            scatters.append(
                pltpu.async_copy(mask_v, out_hbm.at[didx_m.at[b]], sems[b - 1])
            )
        for s in scatters:
            s.wait()

    return body


def kernel(x, emb_table):
    bsz, seqlen = x.shape
    vocab, hidden = emb_table.shape
    t = seqlen + NATIVE_DRAFT_LEN - 1
    b_per_w = bsz // _NW
    midx = jnp.full((8,), MASK_TOKEN_ID, dtype=jnp.int32)
    mdst = (
        jnp.arange(bsz, dtype=jnp.int32)[:, None] * t
        + jnp.minimum(seqlen + jnp.arange(8, dtype=jnp.int32), t - 1)[None, :]
    ).reshape(_NW, b_per_w, 8)
    body = _make_body(bsz, seqlen, hidden)
    out = body(x.astype(jnp.int32), emb_table, midx, mdst)
    return out.reshape(bsz, t, hidden)


# speedup vs baseline: 1.2304x; 1.2304x over previous
import functools

import jax
import jax.numpy as jnp
from jax import lax
from jax.experimental import pallas as pl
from jax.experimental.pallas import tpu as pltpu
from jax.experimental.pallas import tpu_sc as plsc

MASK_TOKEN_ID = 99999
NATIVE_DRAFT_LEN = 8

try:
    _info = plsc.get_sparse_core_info()
    _NC = _info.num_cores
    _NS = _info.num_subcores
except ValueError:
    _NC, _NS = 2, 16
_NW = _NC * _NS


@functools.cache
def _make_body(bsz, seqlen, hidden):
    t = seqlen + NATIVE_DRAFT_LEN - 1
    tp = (t + 7) // 8 * 8
    b_per_w = bsz // _NW
    mesh = plsc.VectorSubcoreMesh(
        core_axis_name="c",
        subcore_axis_name="s",
        num_cores=_NC,
        num_subcores=_NS,
    )
    nsem = 2 * b_per_w + 2

    @functools.partial(
        pl.kernel,
        mesh=mesh,
        out_type=jax.ShapeDtypeStruct((bsz * tp, hidden), jnp.float32),
        scratch_types=[
            pltpu.VMEM((b_per_w, seqlen), jnp.int32),
            pltpu.VMEM((b_per_w, seqlen), jnp.int32),
            pltpu.VMEM((8,), jnp.int32),
            pltpu.VMEM((b_per_w, seqlen), jnp.int32),
            pltpu.VMEM((b_per_w, 8), jnp.int32),
            pltpu.VMEM((b_per_w * seqlen, hidden), jnp.float32),
            pltpu.VMEM((8, hidden), jnp.float32),
            pltpu.VMEM_SHARED((8, hidden), jnp.float32),
            pltpu.SemaphoreType.DMA,
        ]
        + [pltpu.SemaphoreType.DMA] * nsem,
    )
    def body(
        x_hbm, table_hbm, midx_hbm, mdst_hbm, out_hbm,
        x_v, gidx_v, gidx_m, didx_v, didx_m, rows_v, mask_v, mask_s,
        xsem, *sems,
    ):
        sid = lax.axis_index("s")
        wid = sid * _NC + lax.axis_index("c")
        base = wid * b_per_w
        xcopy = pltpu.async_copy(x_hbm.at[pl.ds(base, b_per_w)], x_v, xsem)

        @pl.when(sid == 0)
        def _fetch_mask():
            pltpu.sync_copy(midx_hbm, gidx_m)
            pltpu.async_copy(
                table_hbm.at[gidx_m], mask_v, sems[b_per_w]
            ).wait()
            pltpu.sync_copy(mask_v, mask_s)

        xcopy.wait()
        for b in range(b_per_w):
            gidx_v[b, :] = x_v[b, :]
        g_tok = [
            pltpu.async_copy(
                table_hbm.at[gidx_v.at[b]],
                rows_v.at[pl.ds(b * seqlen, seqlen)],
                sems[b],
            )
            for b in range(b_per_w)
        ]
        iota = lax.iota(jnp.int32, seqlen)
        for b in range(b_per_w):
            didx_v[b, :] = (base + b) * tp + iota
        pltpu.sync_copy(mdst_hbm.at[wid], didx_m)
        plsc.subcore_barrier()

        @pl.when(sid != 0)
        def _pull_mask():
            pltpu.sync_copy(mask_s, mask_v)

        scatters = []
        for b in range(b_per_w):
            g_tok[b].wait()
            scatters.append(
                pltpu.async_copy(
                    rows_v.at[pl.ds(b * seqlen, seqlen)],
                    out_hbm.at[didx_v.at[b]],
                    sems[b_per_w + 1 + b],
                )
            )
        scatters.append(
            pltpu.async_copy(
                mask_v, out_hbm.at[didx_m.at[0]], sems[2 * b_per_w + 1]
            )
        )
        for b in range(1, b_per_w):
            scatters.append(
                pltpu.async_copy(mask_v, out_hbm.at[didx_m.at[b]], sems[b - 1])
            )
        for s in scatters:
            s.wait()

    return body


def kernel(x, emb_table):
    bsz, seqlen = x.shape
    vocab, hidden = emb_table.shape
    t = seqlen + NATIVE_DRAFT_LEN - 1
    b_per_w = bsz // _NW
    tp = (t + 7) // 8 * 8
    midx = jnp.full((8,), MASK_TOKEN_ID, dtype=jnp.int32)
    mdst = (
        jnp.arange(bsz, dtype=jnp.int32)[:, None] * tp
        + (seqlen + jnp.arange(8, dtype=jnp.int32))[None, :]
    ).reshape(_NW, b_per_w, 8)
    body = _make_body(bsz, seqlen, hidden)
    out = body(x.astype(jnp.int32), emb_table, midx, mdst)
    return out.reshape(bsz, tp, hidden)[:, :t, :]
